# Initial kernel scaffold; baseline (speedup 1.0000x reference)
import jax
import jax.numpy as jnp
from jax.experimental import pallas as pl


def kernel(temporal_tokens, lookup_table_bias):
    raise NotImplementedError("write your pallas kernel here")


# speedup vs baseline: 147.3137x; 147.3137x over previous
import functools

import jax
import jax.numpy as jnp
from jax import lax
from jax.experimental import pallas as pl
from jax.experimental.pallas import tpu as pltpu
from jax.experimental.pallas import tpu_sc as plsc

H = 16
NB = 19
L = 2048
NV = 10
LANES = 16

NC, NS = 2, 16
NW = NC * NS
HALVES = NW // H
ROWS = L // HALVES
CHUNK = 16


def _sc_bias_kernel(t_hbm, tab_hbm, out_hbm, t_v, tab_v, r_v, sem):
    wid = lax.axis_index("s") * NC + lax.axis_index("c")
    h = wid // HALVES
    r0 = (wid % HALVES) * ROWS

    pltpu.sync_copy(t_hbm, t_v)
    pltpu.sync_copy(tab_hbm, tab_v)

    hbase = h * NB

    def build(c, carry):
        tv = t_v[pl.ds(c * LANES, LANES)]
        for a in range(NV):
            idx = (hbase + a + NB // 2) - tv
            r_v[a, pl.ds(c * LANES, LANES)] = plsc.load_gather(tab_v, [idx])
        return carry

    lax.fori_loop(0, L // LANES, build, 0)

    iota = lax.iota(jnp.int32, LANES)

    def emit(c, carry):
        row0 = r0 + c * CHUNK
        tv = t_v[pl.ds(row0, CHUNK)]
        copies = []
        for k in range(CHUNK):
            a = jnp.sum(jnp.where(iota == k, tv, 0))
            cp = pltpu.make_async_copy(r_v.at[a], out_hbm.at[h, row0 + k], sem)
            cp.start()
            copies.append(cp)
        for cp in copies:
            cp.wait()
        return carry

    lax.fori_loop(0, ROWS // CHUNK, emit, 0)


def kernel(temporal_tokens, lookup_table_bias):
    t = temporal_tokens[0]

    run = functools.partial(
        pl.kernel,
        out_type=jax.ShapeDtypeStruct((H, L, L), jnp.float32),
        mesh=plsc.VectorSubcoreMesh(core_axis_name="c", subcore_axis_name="s"),
        scratch_types=[
            pltpu.VMEM((L,), jnp.int32),
            pltpu.VMEM((H * NB,), jnp.float32),
            pltpu.VMEM((NV, L), jnp.float32),
            pltpu.SemaphoreType.DMA,
        ],
        compiler_params=pltpu.CompilerParams(needs_layout_passes=False),
    )(_sc_bias_kernel)

    out = run(t, lookup_table_bias.reshape(H * NB))
    return out.reshape(1, H, L, L)
